# Initial kernel scaffold; baseline (speedup 1.0000x reference)
import jax
import jax.numpy as jnp
from jax.experimental import pallas as pl


def kernel(x, theta_t, omega, alive_mask, K, edge_index, sW1, sb1, sW2, sb2, sW3, sb3, pW1, pb1, pW2, pb2, eW1, eb1, eW2, eb2, eW3, eb3, nW1, nb1, nW2, nb2, nW3, nb3):
    raise NotImplementedError("write your pallas kernel here")


# speedup vs baseline: 6.7812x; 6.7812x over previous
import functools

import jax
import jax.numpy as jnp
from jax import lax
from jax.experimental import pallas as pl
from jax.experimental.pallas import tpu as pltpu
from jax.experimental.pallas import tpu_sc as plsc

NC = 2
NS = 16
NW = NC * NS
LANES = 16

HF = 64
HX = 80
CH = 4
BG = CH * 128
BS = 512


def _mm(a, b):
    return jax.lax.dot(a, b, precision=None)


def _t1_body(x_ref, th_ref, sW1, sb1, sW2, sb2, sW3, sb3, pW1, pb1, pW2, pb2,
             out_ref):
    x4 = x_ref[:, 0:4]
    h = jax.nn.relu(_mm(x4, sW1[...]) + sb1[...])
    h = jax.nn.relu(_mm(h, sW2[...]) + sb2[...])
    h = _mm(h, sW3[...]) + sb3[...]
    post = _mm(jax.nn.relu(_mm(h, pW1[...]) + pb1[...]), pW2[...]) + pb2[...]
    h = h + post
    th = th_ref[...]
    pad = jnp.zeros((x_ref.shape[0], HX - HF - 1), jnp.float32)
    out_ref[...] = jnp.concatenate([h, th, pad], axis=1)


def _t1(x, theta, sW1, sb1, sW2, sb2, sW3, sb3, pW1, pb1, pW2, pb2, bn):
    n = x.shape[0]
    grid = n // bn
    full = lambda shp: pl.BlockSpec(shp, lambda i: (0,) * len(shp))
    return pl.pallas_call(
        _t1_body,
        grid=(grid,),
        in_specs=[
            pl.BlockSpec((bn, 10), lambda i: (i, 0)),
            pl.BlockSpec((bn, 1), lambda i: (i, 0)),
            full((4, HF)), full((HF,)), full((HF, HF)), full((HF,)),
            full((HF, HF)), full((HF,)), full((HF, HF)), full((HF,)),
            full((HF, HF)), full((HF,)),
        ],
        out_specs=pl.BlockSpec((bn, HX), lambda i: (i, 0)),
        out_shape=jax.ShapeDtypeStruct((n, HX), jnp.float32),
    )(x, theta, sW1, sb1, sW2, sb2, sW3, sb3, pW1, pb1, pW2, pb2)


def _gather_body(hx_hbm, srcg_hbm, dstg_hbm, srcx_hbm, dstx_hbm,
                 idx_s, idx_d, rows_s, rows_d, sem, *, blocks_per_worker):
    wid = lax.axis_index("s") * NC + lax.axis_index("c")
    row_base = wid * blocks_per_worker * CH
    e_base = wid * blocks_per_worker * BG

    def body(j, carry):
        r0 = row_base + j * CH
        e0 = e_base + j * BG
        pltpu.sync_copy(srcg_hbm.at[pl.ds(r0, CH)], idx_s)
        pltpu.sync_copy(dstg_hbm.at[pl.ds(r0, CH)], idx_d)
        descs = []
        for t in range(CH):
            descs.append(pltpu.async_copy(
                hx_hbm.at[idx_s.at[t]], rows_s.at[pl.ds(t * 128, 128)], sem))
            descs.append(pltpu.async_copy(
                hx_hbm.at[idx_d.at[t]], rows_d.at[pl.ds(t * 128, 128)], sem))
        for dsc in descs:
            dsc.wait()
        pltpu.sync_copy(rows_s, srcx_hbm.at[pl.ds(e0, BG)])
        pltpu.sync_copy(rows_d, dstx_hbm.at[pl.ds(e0, BG)])
        return carry

    lax.fori_loop(0, blocks_per_worker, body, 0)


def _gather(hx, srcg, dstg, epad):
    blocks_per_worker = epad // (NW * BG)
    mesh = plsc.VectorSubcoreMesh(core_axis_name="c", subcore_axis_name="s")
    kfn = pl.kernel(
        functools.partial(_gather_body, blocks_per_worker=blocks_per_worker),
        out_type=[
            jax.ShapeDtypeStruct((epad, HX), jnp.float32),
            jax.ShapeDtypeStruct((epad, HX), jnp.float32),
        ],
        mesh=mesh,
        scratch_types=[
            pltpu.VMEM((CH, 128), jnp.int32),
            pltpu.VMEM((CH, 128), jnp.int32),
            pltpu.VMEM((BG, HX), jnp.float32),
            pltpu.VMEM((BG, HX), jnp.float32),
            pltpu.SemaphoreType.DMA,
        ],
        compiler_params=pltpu.CompilerParams(use_tc_tiling_on_sc=False),
    )
    return kfn(hx, srcg, dstg)


def _t2_body(sx_ref, dx_ref, ksc, eW1f, eb1, eW2, eb2, eW3, eb3,
             out_ref):
    hs = sx_ref[:, 0:HF]
    ths = sx_ref[:, HF:HF + 1]
    hd = dx_ref[:, 0:HF]
    thd = dx_ref[:, HF:HF + 1]
    d = ths - thd
    sin_d = jnp.sin(d)
    cos_d = jnp.cos(d)
    kcol = jnp.broadcast_to(ksc[...], (sx_ref.shape[0], 1))
    ef = jnp.concatenate([hd, hs, sin_d, cos_d, kcol], axis=1)
    z = jax.nn.relu(_mm(ef, eW1f[...]) + eb1[...])
    z = jax.nn.relu(_mm(z, eW2[...]) + eb2[...])
    out_ref[...] = _mm(z, eW3[...]) + eb3[...]


def _t2(srcx, dstx, ksc, eW1f, eb1, eW2, eb2, eW3, eb3, be):
    epad = srcx.shape[0]
    grid = epad // be
    full = lambda shp: pl.BlockSpec(shp, lambda i: (0,) * len(shp))
    return pl.pallas_call(
        _t2_body,
        grid=(grid,),
        in_specs=[
            pl.BlockSpec((be, HX), lambda i: (i, 0)),
            pl.BlockSpec((be, HX), lambda i: (i, 0)),
            full((1, 1)),
            full((2 * HF + 3, HF)), full((HF,)),
            full((HF, HF)), full((HF,)), full((HF, HF)), full((HF,)),
        ],
        out_specs=pl.BlockSpec((be, HF), lambda i: (i, 0)),
        out_shape=jax.ShapeDtypeStruct((epad, HF), jnp.float32),
    )(srcx, dstx, ksc, eW1f, eb1, eW2, eb2, eW3, eb3)


def _scatter_body(msg_hbm, dsts_hbm, agg_hbm, msg_v, idx_v, zbuf, shared, sem,
                  *, base, nh, blocks_per_tile, acc_rows):
    del sem
    cid = lax.axis_index("c")
    sid = lax.axis_index("s")
    lo = base + cid * nh
    lo_out = cid * nh
    trash = nh

    for r in range(16):
        for q in range(HF // LANES):
            zbuf[r, pl.ds(q * LANES, LANES)] = jnp.zeros((LANES,), jnp.float32)
    stripe = acc_rows // NS

    def zbody(j, carry):
        pltpu.sync_copy(zbuf, shared.at[pl.ds(sid * stripe + j * 16, 16)])
        return carry

    lax.fori_loop(0, stripe // 16, zbody, 0)
    plsc.subcore_barrier()

    def body(j, carry):
        e0 = (sid * blocks_per_tile + j) * BS
        r0 = e0 // 128
        pltpu.sync_copy(msg_hbm.at[pl.ds(e0, BS)], msg_v)
        pltpu.sync_copy(dsts_hbm.at[pl.ds(r0, BS // 128)], idx_v)
        for t in range(BS // 128):
            for q in range(128 // LANES):
                v = idx_v[t, pl.ds(q * LANES, LANES)]
                m = (v >= lo) & (v < lo + nh)
                idx_v[t, pl.ds(q * LANES, LANES)] = jnp.where(m, v - lo, trash)
        for t in range(BS // 128):
            pltpu.sync_copy(msg_v.at[pl.ds(t * 128, 128)],
                            shared.at[idx_v.at[t]], add=True)
        return carry

    lax.fori_loop(0, blocks_per_tile, body, 0)
    plsc.subcore_barrier()

    n_full = nh // stripe
    rem = nh - n_full * stripe

    @pl.when(sid < n_full)
    def _():
        pltpu.sync_copy(shared.at[pl.ds(sid * stripe, stripe)],
                        agg_hbm.at[pl.ds(lo_out + sid * stripe, stripe)])

    if rem > 0:
        @pl.when(sid == n_full)
        def _():
            pltpu.sync_copy(shared.at[pl.ds(n_full * stripe, rem)],
                            agg_hbm.at[pl.ds(lo_out + n_full * stripe, rem)])


def _scatter(msg, dsts, n_nodes, base):
    epad = msg.shape[0]
    blocks_per_tile = epad // (NS * BS)
    half = n_nodes // 2
    nh = half // NC
    acc_rows = ((nh + 1 + 255) // 256) * 256
    mesh = plsc.VectorSubcoreMesh(core_axis_name="c", subcore_axis_name="s")
    kfn = pl.kernel(
        functools.partial(_scatter_body, base=base, nh=nh,
                          blocks_per_tile=blocks_per_tile, acc_rows=acc_rows),
        out_type=[jax.ShapeDtypeStruct((half, HF), jnp.float32)],
        mesh=mesh,
        scratch_types=[
            pltpu.VMEM((BS, HF), jnp.float32),
            pltpu.VMEM((BS // 128, 128), jnp.int32),
            pltpu.VMEM((16, HF), jnp.float32),
            pltpu.VMEM_SHARED((acc_rows, HF), jnp.float32),
            pltpu.SemaphoreType.DMA,
        ],
        compiler_params=pltpu.CompilerParams(use_tc_tiling_on_sc=False),
    )
    return kfn(msg, dsts)[0]


def _t3_body(hx_ref, agg_ref, om_ref, al_ref,
             nW1f, nb1, nW2, nb2, nW3, nb3,
             delta_ref, theta_ref):
    h = hx_ref[:, 0:HF]
    th = hx_ref[:, HF:HF + 1]
    om = om_ref[...]
    nf = jnp.concatenate([h, agg_ref[...], om, jnp.sin(th), jnp.cos(th)],
                         axis=1)
    z = jax.nn.relu(_mm(nf, nW1f[...]) + nb1[...])
    z = jax.nn.relu(_mm(z, nW2[...]) + nb2[...])
    delta = (_mm(z, nW3[...]) + nb3[...]) * al_ref[...]
    delta_ref[...] = delta
    tp = th + delta
    theta_ref[...] = jnp.arctan2(jnp.sin(tp), jnp.cos(tp))


def _t3(hx, agg, omega, alive, nW1f, nb1, nW2, nb2, nW3, nb3, bn):
    n = hx.shape[0]
    grid = n // bn
    full = lambda shp: pl.BlockSpec(shp, lambda i: (0,) * len(shp))
    return pl.pallas_call(
        _t3_body,
        grid=(grid,),
        in_specs=[
            pl.BlockSpec((bn, HX), lambda i: (i, 0)),
            pl.BlockSpec((bn, HF), lambda i: (i, 0)),
            pl.BlockSpec((bn, 1), lambda i: (i, 0)),
            pl.BlockSpec((bn, 1), lambda i: (i, 0)),
            full((2 * HF + 3, HF)), full((HF,)),
            full((HF, HF)), full((HF,)), full((HF, 1)), full((1,)),
        ],
        out_specs=[
            pl.BlockSpec((bn, 1), lambda i: (i, 0)),
            pl.BlockSpec((bn, 1), lambda i: (i, 0)),
        ],
        out_shape=[
            jax.ShapeDtypeStruct((n, 1), jnp.float32),
            jax.ShapeDtypeStruct((n, 1), jnp.float32),
        ],
    )(hx, agg, omega, alive, nW1f, nb1, nW2, nb2, nW3, nb3)


def kernel(x, theta_t, omega, alive_mask, K, edge_index,
           sW1, sb1, sW2, sb2, sW3, sb3,
           pW1, pb1, pW2, pb2,
           eW1, eb1, eW2, eb2, eW3, eb3,
           nW1, nb1, nW2, nb2, nW3, nb3):
    n = x.shape[0]
    e = edge_index.shape[1]
    bn = 2000
    assert n % bn == 0

    epad = ((e + NW * BG - 1) // (NW * BG)) * (NW * BG)
    src = edge_index[0]
    dst = edge_index[1]
    pad = epad - e
    src_g = jnp.pad(src, (0, pad)).reshape(epad // 128, 128)
    dst_g = jnp.pad(dst, (0, pad)).reshape(epad // 128, 128)
    dst_s = jnp.pad(dst, (0, pad), constant_values=n).reshape(epad // 128, 128)

    theta2 = theta_t.reshape(n, 1)
    hx = _t1(x, theta2, sW1, sb1, sW2, sb2, sW3, sb3, pW1, pb1, pW2, pb2, bn)

    srcx, dstx = _gather(hx, src_g, dst_g, epad)

    msg = _t2(srcx, dstx, K.reshape(1, 1), eW1, eb1, eW2, eb2, eW3, eb3, 8192)

    agg_lo = _scatter(msg, dst_s, n, 0)
    agg_hi = _scatter(msg, dst_s, n, n // 2)
    agg = jnp.concatenate([agg_lo, agg_hi], axis=0)

    delta2, theta_next2 = _t3(hx, agg, omega.reshape(n, 1),
                              alive_mask.reshape(n, 1),
                              nW1, nb1, nW2, nb2, nW3.reshape(HF, 1),
                              nb3, bn)
    return (delta2.reshape(n), theta_next2.reshape(n))


# speedup vs baseline: 6.7950x; 1.0020x over previous
import functools

import jax
import jax.numpy as jnp
from jax import lax
from jax.experimental import pallas as pl
from jax.experimental.pallas import tpu as pltpu
from jax.experimental.pallas import tpu_sc as plsc

NC = 2
NS = 16
NW = NC * NS
LANES = 16

HF = 64
HX = 80
CH = 4
BG = CH * 128
BS = 512


def _mm(a, b):
    return jax.lax.dot(a, b, precision=None)


def _t1_body(x_ref, th_ref, sW1, sb1, sW2, sb2, sW3, sb3, pW1, pb1, pW2, pb2,
             out_ref):
    x4 = x_ref[:, 0:4]
    h = jax.nn.relu(_mm(x4, sW1[...]) + sb1[...])
    h = jax.nn.relu(_mm(h, sW2[...]) + sb2[...])
    h = _mm(h, sW3[...]) + sb3[...]
    post = _mm(jax.nn.relu(_mm(h, pW1[...]) + pb1[...]), pW2[...]) + pb2[...]
    h = h + post
    th = th_ref[...]
    pad = jnp.zeros((x_ref.shape[0], HX - HF - 1), jnp.float32)
    out_ref[...] = jnp.concatenate([h, th, pad], axis=1)


def _t1(x, theta, sW1, sb1, sW2, sb2, sW3, sb3, pW1, pb1, pW2, pb2, bn):
    n = x.shape[0]
    grid = n // bn
    full = lambda shp: pl.BlockSpec(shp, lambda i: (0,) * len(shp))
    return pl.pallas_call(
        _t1_body,
        grid=(grid,),
        in_specs=[
            pl.BlockSpec((bn, 10), lambda i: (i, 0)),
            pl.BlockSpec((bn, 1), lambda i: (i, 0)),
            full((4, HF)), full((HF,)), full((HF, HF)), full((HF,)),
            full((HF, HF)), full((HF,)), full((HF, HF)), full((HF,)),
            full((HF, HF)), full((HF,)),
        ],
        out_specs=pl.BlockSpec((bn, HX), lambda i: (i, 0)),
        out_shape=jax.ShapeDtypeStruct((n, HX), jnp.float32),
    )(x, theta, sW1, sb1, sW2, sb2, sW3, sb3, pW1, pb1, pW2, pb2)


def _gather_body(hx_hbm, srcg_hbm, dstg_hbm, srcx_hbm, dstx_hbm,
                 idx_s, idx_d, rows_s, rows_d, sem, *, blocks_per_worker):
    wid = lax.axis_index("s") * NC + lax.axis_index("c")
    row_base = wid * blocks_per_worker * CH
    e_base = wid * blocks_per_worker * BG

    def body(j, carry):
        r0 = row_base + j * CH
        e0 = e_base + j * BG
        pltpu.sync_copy(srcg_hbm.at[pl.ds(r0, CH)], idx_s)
        pltpu.sync_copy(dstg_hbm.at[pl.ds(r0, CH)], idx_d)
        descs = []
        for t in range(CH):
            descs.append(pltpu.async_copy(
                hx_hbm.at[idx_s.at[t]], rows_s.at[pl.ds(t * 128, 128)], sem))
            descs.append(pltpu.async_copy(
                hx_hbm.at[idx_d.at[t]], rows_d.at[pl.ds(t * 128, 128)], sem))
        for dsc in descs:
            dsc.wait()
        pltpu.sync_copy(rows_s, srcx_hbm.at[pl.ds(e0, BG)])
        pltpu.sync_copy(rows_d, dstx_hbm.at[pl.ds(e0, BG)])
        return carry

    lax.fori_loop(0, blocks_per_worker, body, 0)


def _gather(hx, srcg, dstg, epad):
    blocks_per_worker = epad // (NW * BG)
    mesh = plsc.VectorSubcoreMesh(core_axis_name="c", subcore_axis_name="s")
    kfn = pl.kernel(
        functools.partial(_gather_body, blocks_per_worker=blocks_per_worker),
        out_type=[
            jax.ShapeDtypeStruct((epad, HX), jnp.float32),
            jax.ShapeDtypeStruct((epad, HX), jnp.float32),
        ],
        mesh=mesh,
        scratch_types=[
            pltpu.VMEM((CH, 128), jnp.int32),
            pltpu.VMEM((CH, 128), jnp.int32),
            pltpu.VMEM((BG, HX), jnp.float32),
            pltpu.VMEM((BG, HX), jnp.float32),
            pltpu.SemaphoreType.DMA,
        ],
        compiler_params=pltpu.CompilerParams(use_tc_tiling_on_sc=False),
    )
    return kfn(hx, srcg, dstg)


def _t2_body(sx_ref, dx_ref, ksc, eW1f, eb1, eW2, eb2, eW3, eb3,
             out_ref):
    hs = sx_ref[:, 0:HF]
    ths = sx_ref[:, HF:HF + 1]
    hd = dx_ref[:, 0:HF]
    thd = dx_ref[:, HF:HF + 1]
    d = ths - thd
    sin_d = jnp.sin(d)
    cos_d = jnp.cos(d)
    kcol = jnp.broadcast_to(ksc[...], (sx_ref.shape[0], 1))
    ef = jnp.concatenate([hd, hs, sin_d, cos_d, kcol], axis=1)
    z = jax.nn.relu(_mm(ef, eW1f[...]) + eb1[...])
    z = jax.nn.relu(_mm(z, eW2[...]) + eb2[...])
    out_ref[...] = _mm(z, eW3[...]) + eb3[...]


def _t2(srcx, dstx, ksc, eW1f, eb1, eW2, eb2, eW3, eb3, be):
    epad = srcx.shape[0]
    grid = epad // be
    full = lambda shp: pl.BlockSpec(shp, lambda i: (0,) * len(shp))
    return pl.pallas_call(
        _t2_body,
        grid=(grid,),
        in_specs=[
            pl.BlockSpec((be, HX), lambda i: (i, 0)),
            pl.BlockSpec((be, HX), lambda i: (i, 0)),
            full((1, 1)),
            full((2 * HF + 3, HF)), full((HF,)),
            full((HF, HF)), full((HF,)), full((HF, HF)), full((HF,)),
        ],
        out_specs=pl.BlockSpec((be, HF), lambda i: (i, 0)),
        out_shape=jax.ShapeDtypeStruct((epad, HF), jnp.float32),
    )(srcx, dstx, ksc, eW1f, eb1, eW2, eb2, eW3, eb3)


def _scatter_body(msg_hbm, dsts_hbm, agg_hbm, msg_v, idx_v, zbuf, shared, sem,
                  sem2, *, base, nh, blocks_per_tile, acc_rows):
    cid = lax.axis_index("c")
    sid = lax.axis_index("s")
    lo = base + cid * nh
    lo_out = cid * nh
    trash = nh

    zr = zbuf.shape[0]
    for r in range(zr):
        for q in range(HF // LANES):
            zbuf[r, pl.ds(q * LANES, LANES)] = jnp.zeros((LANES,), jnp.float32)
    stripe = acc_rows // NS
    n_z = stripe // zr

    def zbody(j, carry):
        pltpu.sync_copy(zbuf, shared.at[pl.ds(sid * stripe + j * zr, zr)])
        return carry

    lax.fori_loop(0, n_z, zbody, 0)
    if stripe % zr:
        pltpu.sync_copy(zbuf.at[pl.ds(0, stripe % zr)],
                        shared.at[pl.ds(sid * stripe + n_z * zr, stripe % zr)])
    plsc.subcore_barrier()

    nb = BS // 128
    ebase = sid * blocks_per_tile * BS

    def load(j, p):
        e0 = ebase + j * BS
        r0 = e0 // 128
        a = pltpu.async_copy(msg_hbm.at[pl.ds(e0, BS)], msg_v.at[p], sem)
        b = pltpu.async_copy(dsts_hbm.at[pl.ds(r0, nb)], idx_v.at[p], sem2)
        return a, b

    def process(p):
        for t in range(nb):
            for q in range(128 // LANES):
                v = idx_v[p, t, pl.ds(q * LANES, LANES)]
                m = (v >= lo) & (v < lo + nh)
                idx_v[p, t, pl.ds(q * LANES, LANES)] = jnp.where(m, v - lo, trash)
        for t in range(nb):
            pltpu.sync_copy(msg_v.at[p].at[pl.ds(t * 128, 128)],
                            shared.at[idx_v.at[p].at[t]], add=True)

    a0, b0 = load(0, 0)

    def body(k, carry):
        j0 = 2 * k
        pltpu.make_async_copy(msg_hbm.at[pl.ds(ebase, BS)], msg_v.at[0], sem).wait()
        pltpu.make_async_copy(dsts_hbm.at[pl.ds(0, nb)], idx_v.at[0], sem2).wait()
        load(j0 + 1, 1)
        process(0)
        pltpu.make_async_copy(msg_hbm.at[pl.ds(ebase, BS)], msg_v.at[1], sem).wait()
        pltpu.make_async_copy(dsts_hbm.at[pl.ds(0, nb)], idx_v.at[1], sem2).wait()

        @pl.when(j0 + 2 < blocks_per_tile)
        def _():
            load(j0 + 2, 0)

        process(1)
        return carry

    lax.fori_loop(0, blocks_per_tile // 2, body, 0)
    plsc.subcore_barrier()

    n_full = nh // stripe
    rem = nh - n_full * stripe

    @pl.when(sid < n_full)
    def _():
        pltpu.sync_copy(shared.at[pl.ds(sid * stripe, stripe)],
                        agg_hbm.at[pl.ds(lo_out + sid * stripe, stripe)])

    if rem > 0:
        @pl.when(sid == n_full)
        def _():
            pltpu.sync_copy(shared.at[pl.ds(n_full * stripe, rem)],
                            agg_hbm.at[pl.ds(lo_out + n_full * stripe, rem)])


def _scatter(msg, dsts, n_nodes, base):
    epad = msg.shape[0]
    blocks_per_tile = epad // (NS * BS)
    half = n_nodes // 2
    nh = half // NC
    acc_rows = ((nh + 1 + 255) // 256) * 256
    mesh = plsc.VectorSubcoreMesh(core_axis_name="c", subcore_axis_name="s")
    kfn = pl.kernel(
        functools.partial(_scatter_body, base=base, nh=nh,
                          blocks_per_tile=blocks_per_tile, acc_rows=acc_rows),
        out_type=[jax.ShapeDtypeStruct((half, HF), jnp.float32)],
        mesh=mesh,
        scratch_types=[
            pltpu.VMEM((2, BS, HF), jnp.float32),
            pltpu.VMEM((2, BS // 128, 128), jnp.int32),
            pltpu.VMEM((112, HF), jnp.float32),
            pltpu.VMEM_SHARED((acc_rows, HF), jnp.float32),
            pltpu.SemaphoreType.DMA,
            pltpu.SemaphoreType.DMA,
        ],
        compiler_params=pltpu.CompilerParams(use_tc_tiling_on_sc=False),
    )
    return kfn(msg, dsts)[0]


def _t3_body(hx_ref, agg_ref, om_ref, al_ref,
             nW1f, nb1, nW2, nb2, nW3, nb3,
             delta_ref, theta_ref):
    h = hx_ref[:, 0:HF]
    th = hx_ref[:, HF:HF + 1]
    om = om_ref[...]
    nf = jnp.concatenate([h, agg_ref[...], om, jnp.sin(th), jnp.cos(th)],
                         axis=1)
    z = jax.nn.relu(_mm(nf, nW1f[...]) + nb1[...])
    z = jax.nn.relu(_mm(z, nW2[...]) + nb2[...])
    delta = (_mm(z, nW3[...]) + nb3[...]) * al_ref[...]
    delta_ref[...] = delta
    tp = th + delta
    theta_ref[...] = jnp.arctan2(jnp.sin(tp), jnp.cos(tp))


def _t3(hx, agg, omega, alive, nW1f, nb1, nW2, nb2, nW3, nb3, bn):
    n = hx.shape[0]
    grid = n // bn
    full = lambda shp: pl.BlockSpec(shp, lambda i: (0,) * len(shp))
    return pl.pallas_call(
        _t3_body,
        grid=(grid,),
        in_specs=[
            pl.BlockSpec((bn, HX), lambda i: (i, 0)),
            pl.BlockSpec((bn, HF), lambda i: (i, 0)),
            pl.BlockSpec((bn, 1), lambda i: (i, 0)),
            pl.BlockSpec((bn, 1), lambda i: (i, 0)),
            full((2 * HF + 3, HF)), full((HF,)),
            full((HF, HF)), full((HF,)), full((HF, 1)), full((1,)),
        ],
        out_specs=[
            pl.BlockSpec((bn, 1), lambda i: (i, 0)),
            pl.BlockSpec((bn, 1), lambda i: (i, 0)),
        ],
        out_shape=[
            jax.ShapeDtypeStruct((n, 1), jnp.float32),
            jax.ShapeDtypeStruct((n, 1), jnp.float32),
        ],
    )(hx, agg, omega, alive, nW1f, nb1, nW2, nb2, nW3, nb3)


def kernel(x, theta_t, omega, alive_mask, K, edge_index,
           sW1, sb1, sW2, sb2, sW3, sb3,
           pW1, pb1, pW2, pb2,
           eW1, eb1, eW2, eb2, eW3, eb3,
           nW1, nb1, nW2, nb2, nW3, nb3):
    n = x.shape[0]
    e = edge_index.shape[1]
    bn = 2000
    assert n % bn == 0

    epad = ((e + NW * BG - 1) // (NW * BG)) * (NW * BG)
    src = edge_index[0]
    dst = edge_index[1]
    pad = epad - e
    src_g = jnp.pad(src, (0, pad)).reshape(epad // 128, 128)
    dst_g = jnp.pad(dst, (0, pad)).reshape(epad // 128, 128)
    dst_s = jnp.pad(dst, (0, pad), constant_values=n).reshape(epad // 128, 128)

    theta2 = theta_t.reshape(n, 1)
    hx = _t1(x, theta2, sW1, sb1, sW2, sb2, sW3, sb3, pW1, pb1, pW2, pb2, bn)

    srcx, dstx = _gather(hx, src_g, dst_g, epad)

    msg = _t2(srcx, dstx, K.reshape(1, 1), eW1, eb1, eW2, eb2, eW3, eb3, 8192)

    agg_lo = _scatter(msg, dst_s, n, 0)
    agg_hi = _scatter(msg, dst_s, n, n // 2)
    agg = jnp.concatenate([agg_lo, agg_hi], axis=0)

    delta2, theta_next2 = _t3(hx, agg, omega.reshape(n, 1),
                              alive_mask.reshape(n, 1),
                              nW1, nb1, nW2, nb2, nW3.reshape(HF, 1),
                              nb3, bn)
    return (delta2.reshape(n), theta_next2.reshape(n))


# speedup vs baseline: 6.7954x; 1.0001x over previous
import functools

import jax
import jax.numpy as jnp
from jax import lax
from jax.experimental import pallas as pl
from jax.experimental.pallas import tpu as pltpu
from jax.experimental.pallas import tpu_sc as plsc

NC = 2
NS = 16
NW = NC * NS
LANES = 16

HF = 64
HX = 80
CH = 4
BG = CH * 128
BS = 512


def _mm(a, b):
    return jax.lax.dot(a, b, precision=None)


def _t1_body(x_ref, th_ref, sW1, sb1, sW2, sb2, sW3, sb3, pW1, pb1, pW2, pb2,
             out_ref):
    x4 = x_ref[:, 0:4]
    h = jax.nn.relu(_mm(x4, sW1[...]) + sb1[...])
    h = jax.nn.relu(_mm(h, sW2[...]) + sb2[...])
    h = _mm(h, sW3[...]) + sb3[...]
    post = _mm(jax.nn.relu(_mm(h, pW1[...]) + pb1[...]), pW2[...]) + pb2[...]
    h = h + post
    th = th_ref[...]
    pad = jnp.zeros((x_ref.shape[0], HX - HF - 1), jnp.float32)
    out_ref[...] = jnp.concatenate([h, th, pad], axis=1)


def _t1(x, theta, sW1, sb1, sW2, sb2, sW3, sb3, pW1, pb1, pW2, pb2, bn):
    n = x.shape[0]
    grid = n // bn
    full = lambda shp: pl.BlockSpec(shp, lambda i: (0,) * len(shp))
    return pl.pallas_call(
        _t1_body,
        grid=(grid,),
        in_specs=[
            pl.BlockSpec((bn, 10), lambda i: (i, 0)),
            pl.BlockSpec((bn, 1), lambda i: (i, 0)),
            full((4, HF)), full((HF,)), full((HF, HF)), full((HF,)),
            full((HF, HF)), full((HF,)), full((HF, HF)), full((HF,)),
            full((HF, HF)), full((HF,)),
        ],
        out_specs=pl.BlockSpec((bn, HX), lambda i: (i, 0)),
        out_shape=jax.ShapeDtypeStruct((n, HX), jnp.float32),
    )(x, theta, sW1, sb1, sW2, sb2, sW3, sb3, pW1, pb1, pW2, pb2)


def _gather_body(hx_hbm, srcg_hbm, dstg_hbm, srcx_hbm, dstx_hbm,
                 idx_s, idx_d, rows_s, rows_d, sem, *, blocks_per_worker):
    wid = lax.axis_index("s") * NC + lax.axis_index("c")
    row_base = wid * blocks_per_worker * CH
    e_base = wid * blocks_per_worker * BG

    def body(j, carry):
        r0 = row_base + j * CH
        e0 = e_base + j * BG
        pltpu.sync_copy(srcg_hbm.at[pl.ds(r0, CH)], idx_s)
        pltpu.sync_copy(dstg_hbm.at[pl.ds(r0, CH)], idx_d)
        descs = []
        for t in range(CH):
            descs.append(pltpu.async_copy(
                hx_hbm.at[idx_s.at[t]], rows_s.at[pl.ds(t * 128, 128)], sem))
            descs.append(pltpu.async_copy(
                hx_hbm.at[idx_d.at[t]], rows_d.at[pl.ds(t * 128, 128)], sem))
        for dsc in descs:
            dsc.wait()
        pltpu.sync_copy(rows_s, srcx_hbm.at[pl.ds(e0, BG)])
        pltpu.sync_copy(rows_d, dstx_hbm.at[pl.ds(e0, BG)])
        return carry

    lax.fori_loop(0, blocks_per_worker, body, 0)


def _gather(hx, srcg, dstg, epad):
    blocks_per_worker = epad // (NW * BG)
    mesh = plsc.VectorSubcoreMesh(core_axis_name="c", subcore_axis_name="s")
    kfn = pl.kernel(
        functools.partial(_gather_body, blocks_per_worker=blocks_per_worker),
        out_type=[
            jax.ShapeDtypeStruct((epad, HX), jnp.float32),
            jax.ShapeDtypeStruct((epad, HX), jnp.float32),
        ],
        mesh=mesh,
        scratch_types=[
            pltpu.VMEM((CH, 128), jnp.int32),
            pltpu.VMEM((CH, 128), jnp.int32),
            pltpu.VMEM((BG, HX), jnp.float32),
            pltpu.VMEM((BG, HX), jnp.float32),
            pltpu.SemaphoreType.DMA,
        ],
        compiler_params=pltpu.CompilerParams(use_tc_tiling_on_sc=False),
    )
    return kfn(hx, srcg, dstg)


def _t2_body(sx_ref, dx_ref, ksc, eW1f, eb1, eW2, eb2, eW3, eb3,
             out_ref):
    hs = sx_ref[:, 0:HF]
    ths = sx_ref[:, HF:HF + 1]
    hd = dx_ref[:, 0:HF]
    thd = dx_ref[:, HF:HF + 1]
    d = ths - thd
    sin_d = jnp.sin(d)
    cos_d = jnp.cos(d)
    kcol = jnp.broadcast_to(ksc[...], (sx_ref.shape[0], 1))
    ef = jnp.concatenate([hd, hs, sin_d, cos_d, kcol], axis=1)
    z = jax.nn.relu(_mm(ef, eW1f[...]) + eb1[...])
    z = jax.nn.relu(_mm(z, eW2[...]) + eb2[...])
    out_ref[...] = _mm(z, eW3[...]) + eb3[...]


def _t2(srcx, dstx, ksc, eW1f, eb1, eW2, eb2, eW3, eb3, be):
    epad = srcx.shape[0]
    grid = epad // be
    full = lambda shp: pl.BlockSpec(shp, lambda i: (0,) * len(shp))
    return pl.pallas_call(
        _t2_body,
        grid=(grid,),
        in_specs=[
            pl.BlockSpec((be, HX), lambda i: (i, 0)),
            pl.BlockSpec((be, HX), lambda i: (i, 0)),
            full((1, 1)),
            full((2 * HF + 3, HF)), full((HF,)),
            full((HF, HF)), full((HF,)), full((HF, HF)), full((HF,)),
        ],
        out_specs=pl.BlockSpec((be, HF), lambda i: (i, 0)),
        out_shape=jax.ShapeDtypeStruct((epad, HF), jnp.float32),
    )(srcx, dstx, ksc, eW1f, eb1, eW2, eb2, eW3, eb3)


def _scatter_body(msg_hbm, dsts_hbm, agg_hbm, msg_v, idx_v, zbuf, shared, sem,
                  sem2, sem3, *, base, nh, blocks_per_tile, acc_rows):
    cid = lax.axis_index("c")
    sid = lax.axis_index("s")
    lo = base + cid * nh
    lo_out = cid * nh
    trash = nh

    zr = zbuf.shape[0]
    for r in range(zr):
        for q in range(HF // LANES):
            zbuf[r, pl.ds(q * LANES, LANES)] = jnp.zeros((LANES,), jnp.float32)
    stripe = acc_rows // NS
    n_z = stripe // zr

    def zbody(j, carry):
        pltpu.sync_copy(zbuf, shared.at[pl.ds(sid * stripe + j * zr, zr)])
        return carry

    lax.fori_loop(0, n_z, zbody, 0)
    if stripe % zr:
        pltpu.sync_copy(zbuf.at[pl.ds(0, stripe % zr)],
                        shared.at[pl.ds(sid * stripe + n_z * zr, stripe % zr)])
    plsc.subcore_barrier()

    nb = BS // 128
    ebase = sid * blocks_per_tile * BS

    def load(j, p):
        e0 = ebase + j * BS
        r0 = e0 // 128
        a = pltpu.async_copy(msg_hbm.at[pl.ds(e0, BS)], msg_v.at[p], sem)
        b = pltpu.async_copy(dsts_hbm.at[pl.ds(r0, nb)], idx_v.at[p], sem2)
        return a, b

    def process(p):
        for t in range(nb):
            for q in range(128 // LANES):
                v = idx_v[p, t, pl.ds(q * LANES, LANES)]
                m = (v >= lo) & (v < lo + nh)
                idx_v[p, t, pl.ds(q * LANES, LANES)] = jnp.where(m, v - lo, trash)
        descs = []
        for t in range(nb):
            descs.append(pltpu.async_copy(
                msg_v.at[p].at[pl.ds(t * 128, 128)],
                shared.at[idx_v.at[p].at[t]], sem3, add=True))
        for dsc in descs:
            dsc.wait()

    a0, b0 = load(0, 0)

    def body(k, carry):
        j0 = 2 * k
        pltpu.make_async_copy(msg_hbm.at[pl.ds(ebase, BS)], msg_v.at[0], sem).wait()
        pltpu.make_async_copy(dsts_hbm.at[pl.ds(0, nb)], idx_v.at[0], sem2).wait()
        load(j0 + 1, 1)
        process(0)
        pltpu.make_async_copy(msg_hbm.at[pl.ds(ebase, BS)], msg_v.at[1], sem).wait()
        pltpu.make_async_copy(dsts_hbm.at[pl.ds(0, nb)], idx_v.at[1], sem2).wait()

        @pl.when(j0 + 2 < blocks_per_tile)
        def _():
            load(j0 + 2, 0)

        process(1)
        return carry

    lax.fori_loop(0, blocks_per_tile // 2, body, 0)
    plsc.subcore_barrier()

    n_full = nh // stripe
    rem = nh - n_full * stripe

    @pl.when(sid < n_full)
    def _():
        pltpu.sync_copy(shared.at[pl.ds(sid * stripe, stripe)],
                        agg_hbm.at[pl.ds(lo_out + sid * stripe, stripe)])

    if rem > 0:
        @pl.when(sid == n_full)
        def _():
            pltpu.sync_copy(shared.at[pl.ds(n_full * stripe, rem)],
                            agg_hbm.at[pl.ds(lo_out + n_full * stripe, rem)])


def _scatter(msg, dsts, n_nodes, base):
    epad = msg.shape[0]
    blocks_per_tile = epad // (NS * BS)
    half = n_nodes // 2
    nh = half // NC
    acc_rows = ((nh + 1 + 255) // 256) * 256
    mesh = plsc.VectorSubcoreMesh(core_axis_name="c", subcore_axis_name="s")
    kfn = pl.kernel(
        functools.partial(_scatter_body, base=base, nh=nh,
                          blocks_per_tile=blocks_per_tile, acc_rows=acc_rows),
        out_type=[jax.ShapeDtypeStruct((half, HF), jnp.float32)],
        mesh=mesh,
        scratch_types=[
            pltpu.VMEM((2, BS, HF), jnp.float32),
            pltpu.VMEM((2, BS // 128, 128), jnp.int32),
            pltpu.VMEM((112, HF), jnp.float32),
            pltpu.VMEM_SHARED((acc_rows, HF), jnp.float32),
            pltpu.SemaphoreType.DMA,
            pltpu.SemaphoreType.DMA,
            pltpu.SemaphoreType.DMA,
        ],
        compiler_params=pltpu.CompilerParams(use_tc_tiling_on_sc=False),
    )
    return kfn(msg, dsts)[0]


def _t3_body(hx_ref, agg_ref, om_ref, al_ref,
             nW1f, nb1, nW2, nb2, nW3, nb3,
             delta_ref, theta_ref):
    h = hx_ref[:, 0:HF]
    th = hx_ref[:, HF:HF + 1]
    om = om_ref[...]
    nf = jnp.concatenate([h, agg_ref[...], om, jnp.sin(th), jnp.cos(th)],
                         axis=1)
    z = jax.nn.relu(_mm(nf, nW1f[...]) + nb1[...])
    z = jax.nn.relu(_mm(z, nW2[...]) + nb2[...])
    delta = (_mm(z, nW3[...]) + nb3[...]) * al_ref[...]
    delta_ref[...] = delta
    tp = th + delta
    theta_ref[...] = jnp.arctan2(jnp.sin(tp), jnp.cos(tp))


def _t3(hx, agg, omega, alive, nW1f, nb1, nW2, nb2, nW3, nb3, bn):
    n = hx.shape[0]
    grid = n // bn
    full = lambda shp: pl.BlockSpec(shp, lambda i: (0,) * len(shp))
    return pl.pallas_call(
        _t3_body,
        grid=(grid,),
        in_specs=[
            pl.BlockSpec((bn, HX), lambda i: (i, 0)),
            pl.BlockSpec((bn, HF), lambda i: (i, 0)),
            pl.BlockSpec((bn, 1), lambda i: (i, 0)),
            pl.BlockSpec((bn, 1), lambda i: (i, 0)),
            full((2 * HF + 3, HF)), full((HF,)),
            full((HF, HF)), full((HF,)), full((HF, 1)), full((1,)),
        ],
        out_specs=[
            pl.BlockSpec((bn, 1), lambda i: (i, 0)),
            pl.BlockSpec((bn, 1), lambda i: (i, 0)),
        ],
        out_shape=[
            jax.ShapeDtypeStruct((n, 1), jnp.float32),
            jax.ShapeDtypeStruct((n, 1), jnp.float32),
        ],
    )(hx, agg, omega, alive, nW1f, nb1, nW2, nb2, nW3, nb3)


def kernel(x, theta_t, omega, alive_mask, K, edge_index,
           sW1, sb1, sW2, sb2, sW3, sb3,
           pW1, pb1, pW2, pb2,
           eW1, eb1, eW2, eb2, eW3, eb3,
           nW1, nb1, nW2, nb2, nW3, nb3):
    n = x.shape[0]
    e = edge_index.shape[1]
    bn = 2000
    assert n % bn == 0

    epad = ((e + NW * BG - 1) // (NW * BG)) * (NW * BG)
    src = edge_index[0]
    dst = edge_index[1]
    pad = epad - e
    src_g = jnp.pad(src, (0, pad)).reshape(epad // 128, 128)
    dst_g = jnp.pad(dst, (0, pad)).reshape(epad // 128, 128)
    dst_s = jnp.pad(dst, (0, pad), constant_values=n).reshape(epad // 128, 128)

    theta2 = theta_t.reshape(n, 1)
    hx = _t1(x, theta2, sW1, sb1, sW2, sb2, sW3, sb3, pW1, pb1, pW2, pb2, bn)

    srcx, dstx = _gather(hx, src_g, dst_g, epad)

    msg = _t2(srcx, dstx, K.reshape(1, 1), eW1, eb1, eW2, eb2, eW3, eb3, 8192)

    agg_lo = _scatter(msg, dst_s, n, 0)
    agg_hi = _scatter(msg, dst_s, n, n // 2)
    agg = jnp.concatenate([agg_lo, agg_hi], axis=0)

    delta2, theta_next2 = _t3(hx, agg, omega.reshape(n, 1),
                              alive_mask.reshape(n, 1),
                              nW1, nb1, nW2, nb2, nW3.reshape(HF, 1),
                              nb3, bn)
    return (delta2.reshape(n), theta_next2.reshape(n))


# speedup vs baseline: 8.3187x; 1.2242x over previous
import functools

import jax
import jax.numpy as jnp
from jax import lax
from jax.experimental import pallas as pl
from jax.experimental.pallas import tpu as pltpu
from jax.experimental.pallas import tpu_sc as plsc

NC = 2
NS = 16
NW = NC * NS
LANES = 16

HF = 64
HX = 80
CH = 4
BG = CH * 128
BS = 512


def _mm(a, b):
    return jax.lax.dot(a, b, precision=None)


def _t1_body(x_ref, th_ref, sW1, sb1, sW2, sb2, sW3, sb3, pW1, pb1, pW2, pb2,
             out_ref):
    x4 = x_ref[:, 0:4]
    h = jax.nn.relu(_mm(x4, sW1[...]) + sb1[...])
    h = jax.nn.relu(_mm(h, sW2[...]) + sb2[...])
    h = _mm(h, sW3[...]) + sb3[...]
    post = _mm(jax.nn.relu(_mm(h, pW1[...]) + pb1[...]), pW2[...]) + pb2[...]
    h = h + post
    th = th_ref[...]
    pad = jnp.zeros((x_ref.shape[0], HX - HF - 1), jnp.float32)
    out_ref[...] = jnp.concatenate([h, th, pad], axis=1)


def _t1(x, theta, sW1, sb1, sW2, sb2, sW3, sb3, pW1, pb1, pW2, pb2, bn):
    n = x.shape[0]
    grid = n // bn
    full = lambda shp: pl.BlockSpec(shp, lambda i: (0,) * len(shp))
    return pl.pallas_call(
        _t1_body,
        grid=(grid,),
        in_specs=[
            pl.BlockSpec((bn, 10), lambda i: (i, 0)),
            pl.BlockSpec((bn, 1), lambda i: (i, 0)),
            full((4, HF)), full((HF,)), full((HF, HF)), full((HF,)),
            full((HF, HF)), full((HF,)), full((HF, HF)), full((HF,)),
            full((HF, HF)), full((HF,)),
        ],
        out_specs=pl.BlockSpec((bn, HX), lambda i: (i, 0)),
        out_shape=jax.ShapeDtypeStruct((n, HX), jnp.float32),
    )(x, theta, sW1, sb1, sW2, sb2, sW3, sb3, pW1, pb1, pW2, pb2)


def _gather_body(hx_hbm, srcg_hbm, dstg_hbm, srcx_hbm, dstx_hbm,
                 idx_s, idx_d, rows_s, rows_d, sem, *, blocks_per_worker):
    wid = lax.axis_index("s") * NC + lax.axis_index("c")
    row_base = wid * blocks_per_worker * CH
    e_base = wid * blocks_per_worker * BG

    def body(j, carry):
        r0 = row_base + j * CH
        e0 = e_base + j * BG
        pltpu.sync_copy(srcg_hbm.at[pl.ds(r0, CH)], idx_s)
        pltpu.sync_copy(dstg_hbm.at[pl.ds(r0, CH)], idx_d)
        descs = []
        for t in range(CH):
            descs.append(pltpu.async_copy(
                hx_hbm.at[idx_s.at[t]], rows_s.at[pl.ds(t * 128, 128)], sem))
            descs.append(pltpu.async_copy(
                hx_hbm.at[idx_d.at[t]], rows_d.at[pl.ds(t * 128, 128)], sem))
        for dsc in descs:
            dsc.wait()
        pltpu.sync_copy(rows_s, srcx_hbm.at[pl.ds(e0, BG)])
        pltpu.sync_copy(rows_d, dstx_hbm.at[pl.ds(e0, BG)])
        return carry

    lax.fori_loop(0, blocks_per_worker, body, 0)


def _gather(hx, srcg, dstg, epad):
    blocks_per_worker = epad // (NW * BG)
    mesh = plsc.VectorSubcoreMesh(core_axis_name="c", subcore_axis_name="s")
    kfn = pl.kernel(
        functools.partial(_gather_body, blocks_per_worker=blocks_per_worker),
        out_type=[
            jax.ShapeDtypeStruct((epad, HX), jnp.float32),
            jax.ShapeDtypeStruct((epad, HX), jnp.float32),
        ],
        mesh=mesh,
        scratch_types=[
            pltpu.VMEM((CH, 128), jnp.int32),
            pltpu.VMEM((CH, 128), jnp.int32),
            pltpu.VMEM((BG, HX), jnp.float32),
            pltpu.VMEM((BG, HX), jnp.float32),
            pltpu.SemaphoreType.DMA,
        ],
        compiler_params=pltpu.CompilerParams(use_tc_tiling_on_sc=False),
    )
    return kfn(hx, srcg, dstg)


def _t2_body(sx_ref, dx_ref, ksc, eW1f, eb1, eW2, eb2, eW3, eb3,
             out_ref):
    hs = sx_ref[:, 0:HF]
    ths = sx_ref[:, HF:HF + 1]
    hd = dx_ref[:, 0:HF]
    thd = dx_ref[:, HF:HF + 1]
    d = ths - thd
    sin_d = jnp.sin(d)
    cos_d = jnp.cos(d)
    kcol = jnp.broadcast_to(ksc[...], (sx_ref.shape[0], 1))
    ef = jnp.concatenate([hd, hs, sin_d, cos_d, kcol], axis=1)
    z = jax.nn.relu(_mm(ef, eW1f[...]) + eb1[...])
    z = jax.nn.relu(_mm(z, eW2[...]) + eb2[...])
    out_ref[...] = _mm(z, eW3[...]) + eb3[...]


def _t2(srcx, dstx, ksc, eW1f, eb1, eW2, eb2, eW3, eb3, be):
    epad = srcx.shape[0]
    grid = epad // be
    full = lambda shp: pl.BlockSpec(shp, lambda i: (0,) * len(shp))
    return pl.pallas_call(
        _t2_body,
        grid=(grid,),
        in_specs=[
            pl.BlockSpec((be, HX), lambda i: (i, 0)),
            pl.BlockSpec((be, HX), lambda i: (i, 0)),
            full((1, 1)),
            full((2 * HF + 3, HF)), full((HF,)),
            full((HF, HF)), full((HF,)), full((HF, HF)), full((HF,)),
        ],
        out_specs=pl.BlockSpec((be, HF), lambda i: (i, 0)),
        out_shape=jax.ShapeDtypeStruct((epad, HF), jnp.float32),
    )(srcx, dstx, ksc, eW1f, eb1, eW2, eb2, eW3, eb3)


def _scatter_body(msg_hbm, dsts_hbm, agg_hbm, msg_v, idx_v, zbuf, shared, sem,
                  sem2, sem3, *, base, nh, blocks_per_tile, acc_rows):
    cid = lax.axis_index("c")
    sid = lax.axis_index("s")
    lo = base + cid * nh
    lo_out = cid * nh
    trash = nh

    zr = zbuf.shape[0]
    for r in range(zr):
        for q in range(HF // LANES):
            zbuf[r, pl.ds(q * LANES, LANES)] = jnp.zeros((LANES,), jnp.float32)
    stripe = acc_rows // NS
    n_z = stripe // zr

    def zbody(j, carry):
        pltpu.sync_copy(zbuf, shared.at[pl.ds(sid * stripe + j * zr, zr)])
        return carry

    lax.fori_loop(0, n_z, zbody, 0)
    if stripe % zr:
        pltpu.sync_copy(zbuf.at[pl.ds(0, stripe % zr)],
                        shared.at[pl.ds(sid * stripe + n_z * zr, stripe % zr)])
    plsc.subcore_barrier()

    nb = BS // 128
    ebase = sid * blocks_per_tile * BS

    def load(j, p):
        e0 = ebase + j * BS
        r0 = e0 // 128
        a = pltpu.async_copy(msg_hbm.at[pl.ds(e0, BS)], msg_v.at[p], sem)
        b = pltpu.async_copy(dsts_hbm.at[pl.ds(r0, nb)], idx_v.at[p], sem2)
        return a, b

    def process(p):
        for t in range(nb):
            for q in range(128 // LANES):
                v = idx_v[p, t, pl.ds(q * LANES, LANES)]
                m = (v >= lo) & (v < lo + nh)
                idx_v[p, t, pl.ds(q * LANES, LANES)] = jnp.where(
                    m, v - lo, trash + (v & 255))
        descs = []
        for t in range(nb):
            descs.append(pltpu.async_copy(
                msg_v.at[p].at[pl.ds(t * 128, 128)],
                shared.at[idx_v.at[p].at[t]], sem3, add=True))
        for dsc in descs:
            dsc.wait()

    a0, b0 = load(0, 0)

    def body(k, carry):
        j0 = 2 * k
        pltpu.make_async_copy(msg_hbm.at[pl.ds(ebase, BS)], msg_v.at[0], sem).wait()
        pltpu.make_async_copy(dsts_hbm.at[pl.ds(0, nb)], idx_v.at[0], sem2).wait()
        load(j0 + 1, 1)
        process(0)
        pltpu.make_async_copy(msg_hbm.at[pl.ds(ebase, BS)], msg_v.at[1], sem).wait()
        pltpu.make_async_copy(dsts_hbm.at[pl.ds(0, nb)], idx_v.at[1], sem2).wait()

        @pl.when(j0 + 2 < blocks_per_tile)
        def _():
            load(j0 + 2, 0)

        process(1)
        return carry

    lax.fori_loop(0, blocks_per_tile // 2, body, 0)
    plsc.subcore_barrier()

    n_full = nh // stripe
    rem = nh - n_full * stripe

    @pl.when(sid < n_full)
    def _():
        pltpu.sync_copy(shared.at[pl.ds(sid * stripe, stripe)],
                        agg_hbm.at[pl.ds(lo_out + sid * stripe, stripe)])

    if rem > 0:
        @pl.when(sid == n_full)
        def _():
            pltpu.sync_copy(shared.at[pl.ds(n_full * stripe, rem)],
                            agg_hbm.at[pl.ds(lo_out + n_full * stripe, rem)])


def _scatter(msg, dsts, n_nodes, base):
    epad = msg.shape[0]
    blocks_per_tile = epad // (NS * BS)
    half = n_nodes // 2
    nh = half // NC
    acc_rows = ((nh + 256 + 255) // 256) * 256
    mesh = plsc.VectorSubcoreMesh(core_axis_name="c", subcore_axis_name="s")
    kfn = pl.kernel(
        functools.partial(_scatter_body, base=base, nh=nh,
                          blocks_per_tile=blocks_per_tile, acc_rows=acc_rows),
        out_type=[jax.ShapeDtypeStruct((half, HF), jnp.float32)],
        mesh=mesh,
        scratch_types=[
            pltpu.VMEM((2, BS, HF), jnp.float32),
            pltpu.VMEM((2, BS // 128, 128), jnp.int32),
            pltpu.VMEM((112, HF), jnp.float32),
            pltpu.VMEM_SHARED((acc_rows, HF), jnp.float32),
            pltpu.SemaphoreType.DMA,
            pltpu.SemaphoreType.DMA,
            pltpu.SemaphoreType.DMA,
        ],
        compiler_params=pltpu.CompilerParams(use_tc_tiling_on_sc=False),
    )
    return kfn(msg, dsts)[0]


def _t3_body(hx_ref, agg_ref, om_ref, al_ref,
             nW1f, nb1, nW2, nb2, nW3, nb3,
             delta_ref, theta_ref):
    h = hx_ref[:, 0:HF]
    th = hx_ref[:, HF:HF + 1]
    om = om_ref[...]
    nf = jnp.concatenate([h, agg_ref[...], om, jnp.sin(th), jnp.cos(th)],
                         axis=1)
    z = jax.nn.relu(_mm(nf, nW1f[...]) + nb1[...])
    z = jax.nn.relu(_mm(z, nW2[...]) + nb2[...])
    delta = (_mm(z, nW3[...]) + nb3[...]) * al_ref[...]
    delta_ref[...] = delta
    tp = th + delta
    theta_ref[...] = jnp.arctan2(jnp.sin(tp), jnp.cos(tp))


def _t3(hx, agg, omega, alive, nW1f, nb1, nW2, nb2, nW3, nb3, bn):
    n = hx.shape[0]
    grid = n // bn
    full = lambda shp: pl.BlockSpec(shp, lambda i: (0,) * len(shp))
    return pl.pallas_call(
        _t3_body,
        grid=(grid,),
        in_specs=[
            pl.BlockSpec((bn, HX), lambda i: (i, 0)),
            pl.BlockSpec((bn, HF), lambda i: (i, 0)),
            pl.BlockSpec((bn, 1), lambda i: (i, 0)),
            pl.BlockSpec((bn, 1), lambda i: (i, 0)),
            full((2 * HF + 3, HF)), full((HF,)),
            full((HF, HF)), full((HF,)), full((HF, 1)), full((1,)),
        ],
        out_specs=[
            pl.BlockSpec((bn, 1), lambda i: (i, 0)),
            pl.BlockSpec((bn, 1), lambda i: (i, 0)),
        ],
        out_shape=[
            jax.ShapeDtypeStruct((n, 1), jnp.float32),
            jax.ShapeDtypeStruct((n, 1), jnp.float32),
        ],
    )(hx, agg, omega, alive, nW1f, nb1, nW2, nb2, nW3, nb3)


def kernel(x, theta_t, omega, alive_mask, K, edge_index,
           sW1, sb1, sW2, sb2, sW3, sb3,
           pW1, pb1, pW2, pb2,
           eW1, eb1, eW2, eb2, eW3, eb3,
           nW1, nb1, nW2, nb2, nW3, nb3):
    n = x.shape[0]
    e = edge_index.shape[1]
    bn = 2000
    assert n % bn == 0

    epad = ((e + NW * BG - 1) // (NW * BG)) * (NW * BG)
    src = edge_index[0]
    dst = edge_index[1]
    pad = epad - e
    src_g = jnp.pad(src, (0, pad)).reshape(epad // 128, 128)
    dst_g = jnp.pad(dst, (0, pad)).reshape(epad // 128, 128)
    dst_s = jnp.pad(dst, (0, pad), constant_values=n).reshape(epad // 128, 128)

    theta2 = theta_t.reshape(n, 1)
    hx = _t1(x, theta2, sW1, sb1, sW2, sb2, sW3, sb3, pW1, pb1, pW2, pb2, bn)

    srcx, dstx = _gather(hx, src_g, dst_g, epad)

    msg = _t2(srcx, dstx, K.reshape(1, 1), eW1, eb1, eW2, eb2, eW3, eb3, 8192)

    agg_lo = _scatter(msg, dst_s, n, 0)
    agg_hi = _scatter(msg, dst_s, n, n // 2)
    agg = jnp.concatenate([agg_lo, agg_hi], axis=0)

    delta2, theta_next2 = _t3(hx, agg, omega.reshape(n, 1),
                              alive_mask.reshape(n, 1),
                              nW1, nb1, nW2, nb2, nW3.reshape(HF, 1),
                              nb3, bn)
    return (delta2.reshape(n), theta_next2.reshape(n))
